# Initial kernel scaffold; baseline (speedup 1.0000x reference)
import jax
import jax.numpy as jnp
from jax.experimental import pallas as pl


def kernel(prediction, label):
    raise NotImplementedError("write your pallas kernel here")


# speedup vs baseline: 13.7299x; 13.7299x over previous
import functools

import jax
import jax.numpy as jnp
from jax import lax
from jax.experimental import pallas as pl
from jax.experimental.pallas import tpu as pltpu
from jax.experimental.pallas import tpu_sc as plsc

N = 16 * 1 * 512 * 512
NKEEP = int(N * 0.8)
NW = 32
PER_W = N // NW
CH = 8192
NCH = PER_W // CH
B = 1024
LMAX = 13.8156
SCALE = B / LMAX
BW = LMAX / B
LN2 = 0.6931471805599453
SQRT2 = 1.4142135623730951


def _log16(x):
    bits = lax.bitcast_convert_type(x, jnp.int32)
    e = lax.shift_right_logical(bits, 23) - 127
    m = lax.bitcast_convert_type(
        jnp.bitwise_or(jnp.bitwise_and(bits, 0x007FFFFF), 0x3F800000),
        jnp.float32)
    big = m > SQRT2
    m = jnp.where(big, m * 0.5, m)
    ef = lax.convert_element_type(e, jnp.float32) + jnp.where(big, 1.0, 0.0)
    s = (m - 1.0) / (m + 1.0)
    s2 = s * s
    poly = 2.0 * s * (1.0 + s2 * (1.0 / 3.0 + s2 * (0.2 + s2 * (1.0 / 7.0))))
    return ef * LN2 + poly


def _loss16(pv, lv):
    lab = lv * 0.95 + 0.05
    p = jnp.clip(pv, 1e-6, 1.0 - 1e-6)
    return -(lab * _log16(p) + (1.0 - lab) * _log16(1.0 - p))


_MESH = plsc.VectorSubcoreMesh(core_axis_name="c", subcore_axis_name="s")


@functools.partial(
    pl.kernel,
    out_type=jax.ShapeDtypeStruct((NW, 2, B), jnp.float32),
    mesh=_MESH,
    compiler_params=pltpu.CompilerParams(needs_layout_passes=False),
    scratch_types=[
        pltpu.VMEM((CH,), jnp.float32),
        pltpu.VMEM((CH,), jnp.float32),
        pltpu.VMEM((B,), jnp.float32),
        pltpu.VMEM((B,), jnp.float32),
    ],
)
def _hist_kernel(pred_hbm, label_hbm, out_hbm, pred_v, label_v, cnt_v, sum_v):
    wid = lax.axis_index("s") * 2 + lax.axis_index("c")
    base = wid * PER_W

    zeros16 = jnp.zeros((16,), jnp.float32)
    ones16 = jnp.ones((16,), jnp.float32)

    def zero_body(i, carry):
        cnt_v[pl.ds(i * 16, 16)] = zeros16
        sum_v[pl.ds(i * 16, 16)] = zeros16
        return carry

    lax.fori_loop(0, B // 16, zero_body, 0)

    def chunk_body(ci, carry):
        off = base + ci * CH
        pltpu.sync_copy(pred_hbm.at[pl.ds(off, CH)], pred_v)
        pltpu.sync_copy(label_hbm.at[pl.ds(off, CH)], label_v)

        def inner(j, c2):
            pv = pred_v[pl.ds(j * 16, 16)]
            lv = label_v[pl.ds(j * 16, 16)]
            loss = _loss16(pv, lv)
            idx = lax.convert_element_type(loss * SCALE, jnp.int32)
            idx = jnp.clip(idx, 0, B - 1)
            plsc.addupdate_scatter(cnt_v, [idx], ones16)
            plsc.addupdate_scatter(sum_v, [idx], loss)
            return c2

        lax.fori_loop(0, CH // 16, inner, 0)
        return carry

    lax.fori_loop(0, NCH, chunk_body, 0)

    pltpu.sync_copy(cnt_v, out_hbm.at[wid, 0])
    pltpu.sync_copy(sum_v, out_hbm.at[wid, 1])


@functools.partial(
    pl.kernel,
    out_type=jax.ShapeDtypeStruct((16,), jnp.float32),
    mesh=_MESH,
    compiler_params=pltpu.CompilerParams(needs_layout_passes=False),
    scratch_types=[
        pltpu.VMEM((B,), jnp.float32),
        pltpu.VMEM((B,), jnp.float32),
        pltpu.VMEM((2, B), jnp.float32),
        pltpu.VMEM((16,), jnp.float32),
    ],
)
def _select_kernel(hist_hbm, out_hbm, acc_c, acc_s, tmp_v, out_v):
    wid = lax.axis_index("s") * 2 + lax.axis_index("c")

    @pl.when(wid == 0)
    def _():
        zeros16 = jnp.zeros((16,), jnp.float32)

        def zero_body(i, carry):
            acc_c[pl.ds(i * 16, 16)] = zeros16
            acc_s[pl.ds(i * 16, 16)] = zeros16
            return carry

        lax.fori_loop(0, B // 16, zero_body, 0)

        def merge_body(t, carry):
            pltpu.sync_copy(hist_hbm.at[t], tmp_v)

            def add_body(i, c2):
                sl = pl.ds(i * 16, 16)
                acc_c[sl] = acc_c[sl] + tmp_v[0, sl]
                acc_s[sl] = acc_s[sl] + tmp_v[1, sl]
                return c2

            lax.fori_loop(0, B // 16, add_body, 0)
            return carry

        lax.fori_loop(0, NW, merge_body, 0)

        nk = jnp.float32(NKEEP)

        def sel_body(jj, carry):
            cum, accv = carry
            j = (B // 16 - 1) - jj
            sl = pl.ds(j * 16, 16)
            cvec = acc_c[sl]
            svec = acc_s[sl]
            sfx = lax.rev(plsc.cumsum(lax.rev(cvec, (0,))), (0,))
            incl = cum + sfx
            above = incl - cvec
            full = incl <= nk
            thr = jnp.logical_and(above < nk, incl > nk)
            safe_c = jnp.maximum(cvec, 1.0)
            meanv = svec / safe_c
            r = nk - above
            vhat = meanv + (1.0 - r / safe_c) * (BW * 0.5)
            contrib = jnp.where(full, svec, 0.0) + jnp.where(thr, r * vhat, 0.0)
            return cum + jnp.sum(cvec), accv + contrib

        _, accv = lax.fori_loop(
            0, B // 16, sel_body,
            (jnp.float32(0.0), jnp.zeros((16,), jnp.float32)))
        total = jnp.sum(accv)
        out_v[...] = jnp.zeros((16,), jnp.float32) + total * (1.0 / NKEEP)
        pltpu.sync_copy(out_v, out_hbm)


def kernel(prediction, label):
    pred = prediction.reshape(-1)
    lab = label.reshape(-1)
    hist = _hist_kernel(pred, lab)
    out = _select_kernel(hist)
    return out[0]


# speedup vs baseline: 17.2289x; 1.2549x over previous
import functools

import jax
import jax.numpy as jnp
from jax import lax
from jax.experimental import pallas as pl
from jax.experimental.pallas import tpu as pltpu
from jax.experimental.pallas import tpu_sc as plsc

N = 16 * 1 * 512 * 512
NKEEP = int(N * 0.8)
NW = 32
PER_W = N // NW
CH = 8192
NCH = PER_W // CH
B = 1024
LMAX = 13.8156
SCALE = B / LMAX
BW = LMAX / B
LN2 = 0.6931471805599453
SQRT2 = 1.4142135623730951

_C1 = 1.0000143715625427
_C2 = -0.4998440549159795
_C3 = 0.3322423278568942
_C4 = -0.2548729786742385
_C5 = 0.2232526535295639
_C6 = -0.14230193464930013


def _log16(x):
    bits = lax.bitcast_convert_type(x, jnp.int32)
    e = lax.shift_right_logical(bits, 23) - 127
    m = lax.bitcast_convert_type(
        jnp.bitwise_or(jnp.bitwise_and(bits, 0x007FFFFF), 0x3F800000),
        jnp.float32)
    big = m > SQRT2
    m = jnp.where(big, m * 0.5, m)
    ef = lax.convert_element_type(e, jnp.float32) + jnp.where(big, 1.0, 0.0)
    t = m - 1.0
    p = _C6
    p = p * t + _C5
    p = p * t + _C4
    p = p * t + _C3
    p = p * t + _C2
    p = p * t + _C1
    return ef * LN2 + p * t


_MESH = plsc.VectorSubcoreMesh(core_axis_name="c", subcore_axis_name="s")


@functools.partial(
    pl.kernel,
    out_type=jax.ShapeDtypeStruct((NW, 2, B), jnp.float32),
    mesh=_MESH,
    compiler_params=pltpu.CompilerParams(needs_layout_passes=False),
    scratch_types=[
        pltpu.VMEM((CH,), jnp.float32),
        pltpu.VMEM((CH,), jnp.float32),
        pltpu.VMEM((CH,), jnp.float32),
        pltpu.VMEM((CH,), jnp.float32),
        pltpu.VMEM((B,), jnp.float32),
        pltpu.VMEM((B,), jnp.float32),
        pltpu.SemaphoreType.DMA,
        pltpu.SemaphoreType.DMA,
    ],
)
def _hist_kernel(pred_hbm, label_hbm, out_hbm,
                 pred0, label0, pred1, label1, cnt_v, sum_v, sem0, sem1):
    wid = lax.axis_index("s") * 2 + lax.axis_index("c")
    base = wid * PER_W

    zeros16 = jnp.zeros((16,), jnp.float32)
    ones16 = jnp.ones((16,), jnp.float32)

    def zero_body(i, carry):
        cnt_v[pl.ds(i * 16, 16)] = zeros16
        sum_v[pl.ds(i * 16, 16)] = zeros16
        return carry

    lax.fori_loop(0, B // 16, zero_body, 0)

    def start(ci, pv, lv, sem):
        off = base + ci * CH
        pltpu.async_copy(pred_hbm.at[pl.ds(off, CH)], pv, sem)
        pltpu.async_copy(label_hbm.at[pl.ds(off, CH)], lv, sem)

    def wait(pv, lv, sem):
        pltpu.make_async_copy(pred_hbm.at[pl.ds(0, CH)], pv, sem).wait()
        pltpu.make_async_copy(label_hbm.at[pl.ds(0, CH)], lv, sem).wait()

    def compute(pv_ref, lv_ref):
        def inner(j, c2):
            for u in range(2):
                sl = pl.ds((2 * j + u) * 16, 16)
                pv = pv_ref[sl]
                lv = lv_ref[sl]
                lab = lv * 0.95 + 0.05
                p = jnp.clip(pv, 1e-6, 1.0 - 1e-6)
                lp = _log16(p)
                lq = _log16(1.0 - p)
                loss = -lq - lab * (lp - lq)
                idx = lax.convert_element_type(loss * SCALE, jnp.int32)
                idx = jnp.clip(idx, 0, B - 1)
                plsc.addupdate_scatter(cnt_v, [idx], ones16)
                plsc.addupdate_scatter(sum_v, [idx], loss)
            return c2

        lax.fori_loop(0, CH // 32, inner, 0)

    start(0, pred0, label0, sem0)

    def pair_body(gi, carry):
        ca = 2 * gi
        wait(pred0, label0, sem0)

        @pl.when(ca + 1 < NCH)
        def _():
            start(ca + 1, pred1, label1, sem1)

        compute(pred0, label0)

        wait(pred1, label1, sem1)

        @pl.when(ca + 2 < NCH)
        def _():
            start(ca + 2, pred0, label0, sem0)

        compute(pred1, label1)
        return carry

    lax.fori_loop(0, NCH // 2, pair_body, 0)

    pltpu.sync_copy(cnt_v, out_hbm.at[wid, 0])
    pltpu.sync_copy(sum_v, out_hbm.at[wid, 1])


@functools.partial(
    pl.kernel,
    out_type=jax.ShapeDtypeStruct((16,), jnp.float32),
    mesh=_MESH,
    compiler_params=pltpu.CompilerParams(needs_layout_passes=False),
    scratch_types=[
        pltpu.VMEM((NW, 2, B), jnp.float32),
        pltpu.VMEM((16,), jnp.float32),
    ],
)
def _select_kernel(hist_hbm, out_hbm, tmp_v, out_v):
    wid = lax.axis_index("s") * 2 + lax.axis_index("c")

    @pl.when(wid == 0)
    def _():
        pltpu.sync_copy(hist_hbm, tmp_v)
        nk = jnp.float32(NKEEP)

        def sel_body(jj, carry):
            cum, accv = carry
            j = (B // 16 - 1) - jj
            sl = pl.ds(j * 16, 16)

            def merge_body(t, cs):
                cvec, svec = cs
                return cvec + tmp_v[t, 0, sl], svec + tmp_v[t, 1, sl]

            cvec, svec = lax.fori_loop(0, NW, merge_body, (zeros16, zeros16))
            sfx = lax.rev(plsc.cumsum(lax.rev(cvec, (0,))), (0,))
            incl = cum + sfx
            above = incl - cvec
            full = incl <= nk
            thr = jnp.logical_and(above < nk, incl > nk)
            safe_c = jnp.maximum(cvec, 1.0)
            meanv = svec / safe_c
            r = nk - above
            vhat = meanv + (1.0 - r / safe_c) * (BW * 0.5)
            contrib = jnp.where(full, svec, 0.0) + jnp.where(thr, r * vhat, 0.0)
            return (cum + jnp.sum(cvec), accv + contrib)

        zeros16 = jnp.zeros((16,), jnp.float32)
        _, accv = lax.fori_loop(
            0, B // 16, sel_body, (jnp.float32(0.0), zeros16))
        total = jnp.sum(accv)
        out_v[...] = jnp.zeros((16,), jnp.float32) + total * (1.0 / NKEEP)
        pltpu.sync_copy(out_v, out_hbm)


def kernel(prediction, label):
    pred = prediction.reshape(-1)
    lab = label.reshape(-1)
    hist = _hist_kernel(pred, lab)
    out = _select_kernel(hist)
    return out[0]


# speedup vs baseline: 17.6510x; 1.0245x over previous
import functools

import jax
import jax.numpy as jnp
from jax import lax
from jax.experimental import pallas as pl
from jax.experimental.pallas import tpu as pltpu
from jax.experimental.pallas import tpu_sc as plsc

N = 16 * 1 * 512 * 512
NKEEP = int(N * 0.8)
NW = 32
PER_W = N // NW
CH = 8192
NCH = PER_W // CH
B = 1024
LMAX = 13.8156
SCALE = B / LMAX
BW = LMAX / B
LN2 = 0.6931471805599453
SQRT2 = 1.4142135623730951

_C1 = 1.0000143715625427
_C2 = -0.4998440549159795
_C3 = 0.3322423278568942
_C4 = -0.2548729786742385
_C5 = 0.2232526535295639
_C6 = -0.14230193464930013


_MAGIC = 0x3F3504F3


def _log16(x):
    bits = lax.bitcast_convert_type(x, jnp.int32)
    e = lax.shift_right_arithmetic(bits - _MAGIC, 23)
    m = lax.bitcast_convert_type(bits - lax.shift_left(e, 23), jnp.float32)
    ef = lax.convert_element_type(e, jnp.float32)
    t = m - 1.0
    p = _C6
    p = p * t + _C5
    p = p * t + _C4
    p = p * t + _C3
    p = p * t + _C2
    p = p * t + _C1
    return ef * LN2 + p * t


_MESH = plsc.VectorSubcoreMesh(core_axis_name="c", subcore_axis_name="s")


@functools.partial(
    pl.kernel,
    out_type=jax.ShapeDtypeStruct((NW, 2, B), jnp.float32),
    mesh=_MESH,
    compiler_params=pltpu.CompilerParams(needs_layout_passes=False),
    scratch_types=[
        pltpu.VMEM((CH,), jnp.float32),
        pltpu.VMEM((CH,), jnp.float32),
        pltpu.VMEM((CH,), jnp.float32),
        pltpu.VMEM((CH,), jnp.float32),
        pltpu.VMEM((B,), jnp.float32),
        pltpu.VMEM((B,), jnp.float32),
        pltpu.SemaphoreType.DMA,
        pltpu.SemaphoreType.DMA,
    ],
)
def _hist_kernel(pred_hbm, label_hbm, out_hbm,
                 pred0, label0, pred1, label1, cnt_v, sum_v, sem0, sem1):
    wid = lax.axis_index("s") * 2 + lax.axis_index("c")
    base = wid * PER_W

    zeros16 = jnp.zeros((16,), jnp.float32)
    ones16 = jnp.ones((16,), jnp.float32)

    def zero_body(i, carry):
        cnt_v[pl.ds(i * 16, 16)] = zeros16
        sum_v[pl.ds(i * 16, 16)] = zeros16
        return carry

    lax.fori_loop(0, B // 16, zero_body, 0)

    def start(ci, pv, lv, sem):
        off = base + ci * CH
        pltpu.async_copy(pred_hbm.at[pl.ds(off, CH)], pv, sem)
        pltpu.async_copy(label_hbm.at[pl.ds(off, CH)], lv, sem)

    def wait(pv, lv, sem):
        pltpu.make_async_copy(pred_hbm.at[pl.ds(0, CH)], pv, sem).wait()
        pltpu.make_async_copy(label_hbm.at[pl.ds(0, CH)], lv, sem).wait()

    def compute(pv_ref, lv_ref):
        def inner(j, c2):
            for u in range(4):
                sl = pl.ds((4 * j + u) * 16, 16)
                pv = pv_ref[sl]
                lv = lv_ref[sl]
                lab = lv * 0.95 + 0.05
                p = jnp.clip(pv, 1e-6, 1.0 - 1e-6)
                lp = _log16(p)
                lq = _log16(1.0 - p)
                loss = -lq - lab * (lp - lq)
                idx = lax.convert_element_type(loss * SCALE, jnp.int32)
                idx = jnp.clip(idx, 0, B - 1)
                plsc.addupdate_scatter(cnt_v, [idx], ones16)
                plsc.addupdate_scatter(sum_v, [idx], loss)
            return c2

        lax.fori_loop(0, CH // 64, inner, 0)

    start(0, pred0, label0, sem0)

    def pair_body(gi, carry):
        ca = 2 * gi
        wait(pred0, label0, sem0)

        @pl.when(ca + 1 < NCH)
        def _():
            start(ca + 1, pred1, label1, sem1)

        compute(pred0, label0)

        wait(pred1, label1, sem1)

        @pl.when(ca + 2 < NCH)
        def _():
            start(ca + 2, pred0, label0, sem0)

        compute(pred1, label1)
        return carry

    lax.fori_loop(0, NCH // 2, pair_body, 0)

    pltpu.sync_copy(cnt_v, out_hbm.at[wid, 0])
    pltpu.sync_copy(sum_v, out_hbm.at[wid, 1])


@functools.partial(
    pl.kernel,
    out_type=jax.ShapeDtypeStruct((16,), jnp.float32),
    mesh=_MESH,
    compiler_params=pltpu.CompilerParams(needs_layout_passes=False),
    scratch_types=[
        pltpu.VMEM((NW, 2, B), jnp.float32),
        pltpu.VMEM((16,), jnp.float32),
    ],
)
def _select_kernel(hist_hbm, out_hbm, tmp_v, out_v):
    wid = lax.axis_index("s") * 2 + lax.axis_index("c")

    @pl.when(wid == 0)
    def _():
        pltpu.sync_copy(hist_hbm, tmp_v)
        nk = jnp.float32(NKEEP)

        def sel_body(jj, carry):
            cum, accv = carry
            j = (B // 16 - 1) - jj
            sl = pl.ds(j * 16, 16)

            def merge_body(t, cs):
                cvec, svec = cs
                return cvec + tmp_v[t, 0, sl], svec + tmp_v[t, 1, sl]

            cvec, svec = lax.fori_loop(0, NW, merge_body, (zeros16, zeros16))
            sfx = lax.rev(plsc.cumsum(lax.rev(cvec, (0,))), (0,))
            incl = cum + sfx
            above = incl - cvec
            full = incl <= nk
            thr = jnp.logical_and(above < nk, incl > nk)
            safe_c = jnp.maximum(cvec, 1.0)
            meanv = svec / safe_c
            r = nk - above
            vhat = meanv + (1.0 - r / safe_c) * (BW * 0.5)
            contrib = jnp.where(full, svec, 0.0) + jnp.where(thr, r * vhat, 0.0)
            return (cum + jnp.sum(cvec), accv + contrib)

        zeros16 = jnp.zeros((16,), jnp.float32)
        _, accv = lax.fori_loop(
            0, B // 16, sel_body, (jnp.float32(0.0), zeros16))
        total = jnp.sum(accv)
        out_v[...] = jnp.zeros((16,), jnp.float32) + total * (1.0 / NKEEP)
        pltpu.sync_copy(out_v, out_hbm)


def kernel(prediction, label):
    pred = prediction.reshape(-1)
    lab = label.reshape(-1)
    hist = _hist_kernel(pred, lab)
    out = _select_kernel(hist)
    return out[0]


# speedup vs baseline: 33.8940x; 1.9202x over previous
import functools

import jax
import jax.numpy as jnp
from jax import lax
from jax.experimental import pallas as pl
from jax.experimental.pallas import tpu as pltpu
from jax.experimental.pallas import tpu_sc as plsc

N = 16 * 1 * 512 * 512
NKEEP = int(N * 0.8)
NW = 32
PER_W = N // NW
CH = 8192
NCH = PER_W // CH
B = 1024
LMAX = 13.8156
SCALE = B / LMAX
BW = LMAX / B
LN2 = 0.6931471805599453
SQRT2 = 1.4142135623730951

_C1 = 1.0000143715625427
_C2 = -0.4998440549159795
_C3 = 0.3322423278568942
_C4 = -0.2548729786742385
_C5 = 0.2232526535295639
_C6 = -0.14230193464930013


_MAGIC = 0x3F3504F3


def _log16_multi(xs):
    bits = [lax.bitcast_convert_type(x, jnp.int32) for x in xs]
    e = [lax.shift_right_arithmetic(b - _MAGIC, 23) for b in bits]
    m = [lax.bitcast_convert_type(b - lax.shift_left(ee, 23), jnp.float32)
         for b, ee in zip(bits, e)]
    ef = [lax.convert_element_type(ee, jnp.float32) for ee in e]
    t = [mm - 1.0 for mm in m]
    p = [jnp.full((16,), _C6, jnp.float32) for _ in xs]
    for c in (_C5, _C4, _C3, _C2, _C1):
        p = [pp * tt + c for pp, tt in zip(p, t)]
    return [f * LN2 + pp * tt for f, pp, tt in zip(ef, p, t)]


_MESH = plsc.VectorSubcoreMesh(core_axis_name="c", subcore_axis_name="s")


@functools.partial(
    pl.kernel,
    out_type=jax.ShapeDtypeStruct((NW, 2, B), jnp.float32),
    mesh=_MESH,
    compiler_params=pltpu.CompilerParams(needs_layout_passes=False),
    scratch_types=[
        pltpu.VMEM((CH,), jnp.float32),
        pltpu.VMEM((CH,), jnp.float32),
        pltpu.VMEM((CH,), jnp.float32),
        pltpu.VMEM((CH,), jnp.float32),
        pltpu.VMEM((B,), jnp.float32),
        pltpu.VMEM((B,), jnp.float32),
        pltpu.SemaphoreType.DMA,
        pltpu.SemaphoreType.DMA,
    ],
)
def _hist_kernel(pred_hbm, label_hbm, out_hbm,
                 pred0, label0, pred1, label1, cnt_v, sum_v, sem0, sem1):
    wid = lax.axis_index("s") * 2 + lax.axis_index("c")
    base = wid * PER_W

    zeros16 = jnp.zeros((16,), jnp.float32)
    ones16 = jnp.ones((16,), jnp.float32)

    def zero_body(i, carry):
        cnt_v[pl.ds(i * 16, 16)] = zeros16
        sum_v[pl.ds(i * 16, 16)] = zeros16
        return carry

    lax.fori_loop(0, B // 16, zero_body, 0)

    def start(ci, pv, lv, sem):
        off = base + ci * CH
        pltpu.async_copy(pred_hbm.at[pl.ds(off, CH)], pv, sem)
        pltpu.async_copy(label_hbm.at[pl.ds(off, CH)], lv, sem)

    def wait(pv, lv, sem):
        pltpu.make_async_copy(pred_hbm.at[pl.ds(0, CH)], pv, sem).wait()
        pltpu.make_async_copy(label_hbm.at[pl.ds(0, CH)], lv, sem).wait()

    def compute(pv_ref, lv_ref):
        U = 4

        def inner(j, c2):
            sls = [pl.ds((U * j + u) * 16, 16) for u in range(U)]
            labs = [lv_ref[sl] * 0.95 + 0.05 for sl in sls]
            ps = [jnp.minimum(jnp.maximum(pv_ref[sl], 1e-6), 1.0 - 1e-6)
                  for sl in sls]
            qs = [1.0 - p for p in ps]
            logs = _log16_multi(ps + qs)
            lps, lqs = logs[:U], logs[U:]
            losses = [-lq - lab * (lp - lq)
                      for lab, lp, lq in zip(labs, lps, lqs)]
            idxs = [lax.convert_element_type(l * SCALE, jnp.int32)
                    for l in losses]
            for idx, l in zip(idxs, losses):
                plsc.addupdate_scatter(cnt_v, [idx], ones16)
                plsc.addupdate_scatter(sum_v, [idx], l)
            return c2

        lax.fori_loop(0, CH // (16 * U), inner, 0)

    start(0, pred0, label0, sem0)

    def pair_body(gi, carry):
        ca = 2 * gi
        wait(pred0, label0, sem0)

        @pl.when(ca + 1 < NCH)
        def _():
            start(ca + 1, pred1, label1, sem1)

        compute(pred0, label0)

        wait(pred1, label1, sem1)

        @pl.when(ca + 2 < NCH)
        def _():
            start(ca + 2, pred0, label0, sem0)

        compute(pred1, label1)
        return carry

    lax.fori_loop(0, NCH // 2, pair_body, 0)

    pltpu.sync_copy(cnt_v, out_hbm.at[wid, 0])
    pltpu.sync_copy(sum_v, out_hbm.at[wid, 1])


@functools.partial(
    pl.kernel,
    out_type=jax.ShapeDtypeStruct((16,), jnp.float32),
    mesh=_MESH,
    compiler_params=pltpu.CompilerParams(needs_layout_passes=False),
    scratch_types=[
        pltpu.VMEM((NW, 2, B), jnp.float32),
        pltpu.VMEM((16,), jnp.float32),
    ],
)
def _select_kernel(hist_hbm, out_hbm, tmp_v, out_v):
    wid = lax.axis_index("s") * 2 + lax.axis_index("c")

    @pl.when(wid == 0)
    def _():
        pltpu.sync_copy(hist_hbm, tmp_v)
        nk = jnp.float32(NKEEP)

        def sel_body(jj, carry):
            cum, accv = carry
            j = (B // 16 - 1) - jj
            sl = pl.ds(j * 16, 16)

            def merge_body(t, cs):
                cvec, svec = cs
                return cvec + tmp_v[t, 0, sl], svec + tmp_v[t, 1, sl]

            cvec, svec = lax.fori_loop(0, NW, merge_body, (zeros16, zeros16))
            sfx = lax.rev(plsc.cumsum(lax.rev(cvec, (0,))), (0,))
            incl = cum + sfx
            above = incl - cvec
            full = incl <= nk
            thr = jnp.logical_and(above < nk, incl > nk)
            safe_c = jnp.maximum(cvec, 1.0)
            meanv = svec / safe_c
            r = nk - above
            vhat = meanv + (1.0 - r / safe_c) * (BW * 0.5)
            contrib = jnp.where(full, svec, 0.0) + jnp.where(thr, r * vhat, 0.0)
            return (cum + jnp.sum(cvec), accv + contrib)

        zeros16 = jnp.zeros((16,), jnp.float32)
        _, accv = lax.fori_loop(
            0, B // 16, sel_body, (jnp.float32(0.0), zeros16))
        total = jnp.sum(accv)
        out_v[...] = jnp.zeros((16,), jnp.float32) + total * (1.0 / NKEEP)
        pltpu.sync_copy(out_v, out_hbm)


def kernel(prediction, label):
    pred = prediction.reshape(-1)
    lab = label.reshape(-1)
    hist = _hist_kernel(pred, lab)
    out = _select_kernel(hist)
    return out[0]


# speedup vs baseline: 35.7224x; 1.0539x over previous
import functools

import jax
import jax.numpy as jnp
from jax import lax
from jax.experimental import pallas as pl
from jax.experimental.pallas import tpu as pltpu
from jax.experimental.pallas import tpu_sc as plsc

N = 16 * 1 * 512 * 512
NKEEP = int(N * 0.8)
NW = 32
PER_W = N // NW
CH = 8192
NCH = PER_W // CH
B = 1024
LMAX = 13.8156
SCALE = B / LMAX
BW = LMAX / B
LN2 = 0.6931471805599453
SQRT2 = 1.4142135623730951

_C1 = 1.0000143715625427
_C2 = -0.4998440549159795
_C3 = 0.3322423278568942
_C4 = -0.2548729786742385
_C5 = 0.2232526535295639
_C6 = -0.14230193464930013


_MAGIC = 0x3F3504F3


def _log16_multi(xs):
    bits = [lax.bitcast_convert_type(x, jnp.int32) for x in xs]
    e = [lax.shift_right_arithmetic(b - _MAGIC, 23) for b in bits]
    m = [lax.bitcast_convert_type(b - lax.shift_left(ee, 23), jnp.float32)
         for b, ee in zip(bits, e)]
    ef = [lax.convert_element_type(ee, jnp.float32) for ee in e]
    t = [mm - 1.0 for mm in m]
    p = [jnp.full((16,), _C6, jnp.float32) for _ in xs]
    for c in (_C5, _C4, _C3, _C2, _C1):
        p = [pp * tt + c for pp, tt in zip(p, t)]
    return [f * LN2 + pp * tt for f, pp, tt in zip(ef, p, t)]


_MESH = plsc.VectorSubcoreMesh(core_axis_name="c", subcore_axis_name="s")


@functools.partial(
    pl.kernel,
    out_type=jax.ShapeDtypeStruct((NW, 2, B), jnp.float32),
    mesh=_MESH,
    compiler_params=pltpu.CompilerParams(needs_layout_passes=False),
    scratch_types=[
        pltpu.VMEM((CH,), jnp.float32),
        pltpu.VMEM((CH,), jnp.float32),
        pltpu.VMEM((CH,), jnp.float32),
        pltpu.VMEM((CH,), jnp.float32),
        pltpu.VMEM((B,), jnp.float32),
        pltpu.VMEM((B,), jnp.float32),
        pltpu.SemaphoreType.DMA,
        pltpu.SemaphoreType.DMA,
    ],
)
def _hist_kernel(pred_hbm, label_hbm, out_hbm,
                 pred0, label0, pred1, label1, cnt_v, sum_v, sem0, sem1):
    wid = lax.axis_index("s") * 2 + lax.axis_index("c")
    base = wid * PER_W

    zeros16 = jnp.zeros((16,), jnp.float32)
    ones16 = jnp.ones((16,), jnp.float32)

    def zero_body(i, carry):
        cnt_v[pl.ds(i * 16, 16)] = zeros16
        sum_v[pl.ds(i * 16, 16)] = zeros16
        return carry

    lax.fori_loop(0, B // 16, zero_body, 0)

    def start(ci, pv, lv, sem):
        off = base + ci * CH
        pltpu.async_copy(pred_hbm.at[pl.ds(off, CH)], pv, sem)
        pltpu.async_copy(label_hbm.at[pl.ds(off, CH)], lv, sem)

    def wait(pv, lv, sem):
        pltpu.make_async_copy(pred_hbm.at[pl.ds(0, CH)], pv, sem).wait()
        pltpu.make_async_copy(label_hbm.at[pl.ds(0, CH)], lv, sem).wait()

    def compute(pv_ref, lv_ref):
        U = 8

        def inner(j, c2):
            sls = [pl.ds((U * j + u) * 16, 16) for u in range(U)]
            labs = [lv_ref[sl] * 0.95 + 0.05 for sl in sls]
            ps = [jnp.minimum(jnp.maximum(pv_ref[sl], 1e-6), 1.0 - 1e-6)
                  for sl in sls]
            qs = [1.0 - p for p in ps]
            logs = _log16_multi(ps + qs)
            lps, lqs = logs[:U], logs[U:]
            losses = [-lq - lab * (lp - lq)
                      for lab, lp, lq in zip(labs, lps, lqs)]
            idxs = [lax.convert_element_type(l * SCALE, jnp.int32)
                    for l in losses]
            for idx, l in zip(idxs, losses):
                plsc.addupdate_scatter(cnt_v, [idx], ones16)
                plsc.addupdate_scatter(sum_v, [idx], l)
            return c2

        lax.fori_loop(0, CH // (16 * U), inner, 0)

    start(0, pred0, label0, sem0)

    def pair_body(gi, carry):
        ca = 2 * gi
        wait(pred0, label0, sem0)

        @pl.when(ca + 1 < NCH)
        def _():
            start(ca + 1, pred1, label1, sem1)

        compute(pred0, label0)

        wait(pred1, label1, sem1)

        @pl.when(ca + 2 < NCH)
        def _():
            start(ca + 2, pred0, label0, sem0)

        compute(pred1, label1)
        return carry

    lax.fori_loop(0, NCH // 2, pair_body, 0)

    pltpu.sync_copy(cnt_v, out_hbm.at[wid, 0])
    pltpu.sync_copy(sum_v, out_hbm.at[wid, 1])


@functools.partial(
    pl.kernel,
    out_type=jax.ShapeDtypeStruct((16,), jnp.float32),
    mesh=_MESH,
    compiler_params=pltpu.CompilerParams(needs_layout_passes=False),
    scratch_types=[
        pltpu.VMEM((NW, 2, B), jnp.float32),
        pltpu.VMEM((16,), jnp.float32),
    ],
)
def _select_kernel(hist_hbm, out_hbm, tmp_v, out_v):
    wid = lax.axis_index("s") * 2 + lax.axis_index("c")

    @pl.when(wid == 0)
    def _():
        pltpu.sync_copy(hist_hbm, tmp_v)
        nk = jnp.float32(NKEEP)

        def sel_body(jj, carry):
            cum, accv = carry
            j = (B // 16 - 1) - jj
            sl = pl.ds(j * 16, 16)

            def merge_body(t, cs):
                cvec, svec = cs
                return cvec + tmp_v[t, 0, sl], svec + tmp_v[t, 1, sl]

            cvec, svec = lax.fori_loop(0, NW, merge_body, (zeros16, zeros16))
            sfx = lax.rev(plsc.cumsum(lax.rev(cvec, (0,))), (0,))
            incl = cum + sfx
            above = incl - cvec
            full = incl <= nk
            thr = jnp.logical_and(above < nk, incl > nk)
            safe_c = jnp.maximum(cvec, 1.0)
            meanv = svec / safe_c
            r = nk - above
            vhat = meanv + (1.0 - r / safe_c) * (BW * 0.5)
            contrib = jnp.where(full, svec, 0.0) + jnp.where(thr, r * vhat, 0.0)
            return (cum + jnp.sum(cvec), accv + contrib)

        zeros16 = jnp.zeros((16,), jnp.float32)
        _, accv = lax.fori_loop(
            0, B // 16, sel_body, (jnp.float32(0.0), zeros16))
        total = jnp.sum(accv)
        out_v[...] = jnp.zeros((16,), jnp.float32) + total * (1.0 / NKEEP)
        pltpu.sync_copy(out_v, out_hbm)


def kernel(prediction, label):
    pred = prediction.reshape(-1)
    lab = label.reshape(-1)
    hist = _hist_kernel(pred, lab)
    out = _select_kernel(hist)
    return out[0]


# speedup vs baseline: 55.7715x; 1.5612x over previous
import functools

import jax
import jax.numpy as jnp
from jax import lax
from jax.experimental import pallas as pl
from jax.experimental.pallas import tpu as pltpu
from jax.experimental.pallas import tpu_sc as plsc

N = 16 * 1 * 512 * 512
NKEEP = int(N * 0.8)
NW = 32
PER_W = N // NW
CH = 8192
NCH = PER_W // CH
B = 1024
LMAX = 13.8156
SCALE = B / LMAX
BW = LMAX / B

ROWS = N // 512
RB = 128


def _loss_body(p_ref, l_ref, o_ref):
    lab = l_ref[...] * 0.95 + 0.05
    p = jnp.clip(p_ref[...], 1e-6, 1.0 - 1e-6)
    loss = -(lab * jnp.log(p) + (1.0 - lab) * jnp.log1p(-p))
    o_ref[...] = loss.reshape(RB, 512)


_loss_tc = pl.pallas_call(
    _loss_body,
    grid=(16, 512 // RB),
    in_specs=[
        pl.BlockSpec((1, 1, RB, 512), lambda b, r: (b, 0, r, 0)),
        pl.BlockSpec((1, 1, RB, 512), lambda b, r: (b, 0, r, 0)),
    ],
    out_specs=pl.BlockSpec((RB, 512), lambda b, r: (b * (512 // RB) + r, 0)),
    out_shape=jax.ShapeDtypeStruct((ROWS, 512), jnp.float32),
    compiler_params=pltpu.CompilerParams(
        dimension_semantics=("parallel", "parallel")),
)

_MESH = plsc.VectorSubcoreMesh(core_axis_name="c", subcore_axis_name="s")


@functools.partial(
    pl.kernel,
    out_type=jax.ShapeDtypeStruct((NW, 2, B), jnp.float32),
    mesh=_MESH,
    compiler_params=pltpu.CompilerParams(needs_layout_passes=False),
    scratch_types=[
        pltpu.VMEM((16, 512), jnp.float32),
        pltpu.VMEM((16, 512), jnp.float32),
        pltpu.VMEM((B,), jnp.float32),
        pltpu.VMEM((B,), jnp.float32),
        pltpu.SemaphoreType.DMA,
        pltpu.SemaphoreType.DMA,
    ],
)
def _hist_kernel(loss_hbm, out_hbm, buf0, buf1, cnt_v, sum_v, sem0, sem1):
    wid = lax.axis_index("s") * 2 + lax.axis_index("c")
    row_base = wid * (PER_W // 512)

    zeros16 = jnp.zeros((16,), jnp.float32)
    ones16 = jnp.ones((16,), jnp.float32)

    def zero_body(i, carry):
        cnt_v[pl.ds(i * 16, 16)] = zeros16
        sum_v[pl.ds(i * 16, 16)] = zeros16
        return carry

    lax.fori_loop(0, B // 16, zero_body, 0)

    def start(ci, buf, sem):
        pltpu.async_copy(
            loss_hbm.at[pl.ds(row_base + ci * 16, 16), :], buf, sem)

    def wait(buf, sem):
        pltpu.make_async_copy(
            loss_hbm.at[pl.ds(0, 16), :], buf, sem).wait()

    def compute(buf):
        U = 8

        def inner(j, c2):
            r = lax.div(j, 4)
            cb = lax.rem(j, 4) * 128
            losses = [buf[r, pl.ds(cb + u * 16, 16)] for u in range(U)]
            idxs = [lax.convert_element_type(l * SCALE, jnp.int32)
                    for l in losses]
            for idx, l in zip(idxs, losses):
                plsc.addupdate_scatter(cnt_v, [idx], ones16)
                plsc.addupdate_scatter(sum_v, [idx], l)
            return c2

        lax.fori_loop(0, CH // (16 * U), inner, 0)

    start(0, buf0, sem0)

    def pair_body(gi, carry):
        ca = 2 * gi
        wait(buf0, sem0)

        @pl.when(ca + 1 < NCH)
        def _():
            start(ca + 1, buf1, sem1)

        compute(buf0)

        wait(buf1, sem1)

        @pl.when(ca + 2 < NCH)
        def _():
            start(ca + 2, buf0, sem0)

        compute(buf1)
        return carry

    lax.fori_loop(0, NCH // 2, pair_body, 0)

    pltpu.sync_copy(cnt_v, out_hbm.at[wid, 0])
    pltpu.sync_copy(sum_v, out_hbm.at[wid, 1])


@functools.partial(
    pl.kernel,
    out_type=jax.ShapeDtypeStruct((16,), jnp.float32),
    mesh=_MESH,
    compiler_params=pltpu.CompilerParams(needs_layout_passes=False),
    scratch_types=[
        pltpu.VMEM((NW, 2, B), jnp.float32),
        pltpu.VMEM((16,), jnp.float32),
    ],
)
def _select_kernel(hist_hbm, out_hbm, tmp_v, out_v):
    wid = lax.axis_index("s") * 2 + lax.axis_index("c")

    @pl.when(wid == 0)
    def _():
        pltpu.sync_copy(hist_hbm, tmp_v)
        nk = jnp.float32(NKEEP)
        zeros16 = jnp.zeros((16,), jnp.float32)

        def sel_body(jj, carry):
            cum, accv = carry
            j = (B // 16 - 1) - jj
            sl = pl.ds(j * 16, 16)

            def merge_body(t, cs):
                cvec, svec = cs
                return cvec + tmp_v[t, 0, sl], svec + tmp_v[t, 1, sl]

            cvec, svec = lax.fori_loop(0, NW, merge_body, (zeros16, zeros16))
            sfx = lax.rev(plsc.cumsum(lax.rev(cvec, (0,))), (0,))
            incl = cum + sfx
            above = incl - cvec
            full = incl <= nk
            thr = jnp.logical_and(above < nk, incl > nk)
            safe_c = jnp.maximum(cvec, 1.0)
            meanv = svec / safe_c
            r = nk - above
            vhat = meanv + (1.0 - r / safe_c) * (BW * 0.5)
            contrib = jnp.where(full, svec, 0.0) + jnp.where(thr, r * vhat, 0.0)
            return (cum + jnp.sum(cvec), accv + contrib)

        _, accv = lax.fori_loop(
            0, B // 16, sel_body, (jnp.float32(0.0), zeros16))
        total = jnp.sum(accv)
        out_v[...] = jnp.zeros((16,), jnp.float32) + total * (1.0 / NKEEP)
        pltpu.sync_copy(out_v, out_hbm)


def kernel(prediction, label):
    loss2d = _loss_tc(prediction, label)
    hist = _hist_kernel(loss2d)
    out = _select_kernel(hist)
    return out[0]


# speedup vs baseline: 56.6474x; 1.0157x over previous
import functools

import jax
import jax.numpy as jnp
from jax import lax
from jax.experimental import pallas as pl
from jax.experimental.pallas import tpu as pltpu
from jax.experimental.pallas import tpu_sc as plsc

N = 16 * 1 * 512 * 512
NKEEP = int(N * 0.8)
NW = 32
PER_W = N // NW
CH = 8192
NCH = PER_W // CH
B = 1024
LMAX = 13.8156
SCALE = B / LMAX
BW = LMAX / B

ROWS = N // 512
RB = 128


def _loss_body(p_ref, l_ref, o_ref):
    lab = l_ref[...] * (0.95 * SCALE) + (0.05 * SCALE)
    p = jnp.clip(p_ref[...], 1e-6, 1.0 - 1e-6)
    lq = jnp.log1p(-p)
    u = -lq * SCALE - lab * (jnp.log(p) - lq)
    o_ref[...] = u.reshape(RB, 512)


_loss_tc = pl.pallas_call(
    _loss_body,
    grid=(16, 512 // RB),
    in_specs=[
        pl.BlockSpec((1, 1, RB, 512), lambda b, r: (b, 0, r, 0)),
        pl.BlockSpec((1, 1, RB, 512), lambda b, r: (b, 0, r, 0)),
    ],
    out_specs=pl.BlockSpec((RB, 512), lambda b, r: (b * (512 // RB) + r, 0)),
    out_shape=jax.ShapeDtypeStruct((ROWS, 512), jnp.float32),
    compiler_params=pltpu.CompilerParams(
        dimension_semantics=("parallel", "parallel")),
)

_MESH = plsc.VectorSubcoreMesh(core_axis_name="c", subcore_axis_name="s")


@functools.partial(
    pl.kernel,
    out_type=jax.ShapeDtypeStruct((NW, 2, B), jnp.float32),
    mesh=_MESH,
    compiler_params=pltpu.CompilerParams(needs_layout_passes=False),
    scratch_types=[
        pltpu.VMEM((16, 512), jnp.float32),
        pltpu.VMEM((16, 512), jnp.float32),
        pltpu.VMEM((B,), jnp.float32),
        pltpu.VMEM((B,), jnp.float32),
        pltpu.SemaphoreType.DMA,
        pltpu.SemaphoreType.DMA,
    ],
)
def _hist_kernel(loss_hbm, out_hbm, buf0, buf1, cnt_v, sum_v, sem0, sem1):
    wid = lax.axis_index("s") * 2 + lax.axis_index("c")
    row_base = wid * (PER_W // 512)

    zeros16 = jnp.zeros((16,), jnp.float32)
    ones16 = jnp.ones((16,), jnp.float32)

    def zero_body(i, carry):
        cnt_v[pl.ds(i * 16, 16)] = zeros16
        sum_v[pl.ds(i * 16, 16)] = zeros16
        return carry

    lax.fori_loop(0, B // 16, zero_body, 0)

    def start(ci, buf, sem):
        pltpu.async_copy(
            loss_hbm.at[pl.ds(row_base + ci * 16, 16), :], buf, sem)

    def wait(buf, sem):
        pltpu.make_async_copy(
            loss_hbm.at[pl.ds(0, 16), :], buf, sem).wait()

    def compute(buf):
        U = 8

        def inner(j, c2):
            r = lax.div(j, 4)
            cb = lax.rem(j, 4) * 128
            losses = [buf[r, pl.ds(cb + u * 16, 16)] for u in range(U)]
            idxs = [lax.convert_element_type(l, jnp.int32) for l in losses]
            for idx, l in zip(idxs, losses):
                plsc.addupdate_scatter(cnt_v, [idx], ones16)
                plsc.addupdate_scatter(sum_v, [idx], l)
            return c2

        lax.fori_loop(0, CH // (16 * U), inner, 0)

    start(0, buf0, sem0)

    def pair_body(gi, carry):
        ca = 2 * gi
        wait(buf0, sem0)

        @pl.when(ca + 1 < NCH)
        def _():
            start(ca + 1, buf1, sem1)

        compute(buf0)

        wait(buf1, sem1)

        @pl.when(ca + 2 < NCH)
        def _():
            start(ca + 2, buf0, sem0)

        compute(buf1)
        return carry

    lax.fori_loop(0, NCH // 2, pair_body, 0)

    pltpu.sync_copy(cnt_v, out_hbm.at[wid, 0])
    pltpu.sync_copy(sum_v, out_hbm.at[wid, 1])


@functools.partial(
    pl.kernel,
    out_type=jax.ShapeDtypeStruct((16,), jnp.float32),
    mesh=_MESH,
    compiler_params=pltpu.CompilerParams(needs_layout_passes=False),
    scratch_types=[
        pltpu.VMEM((NW, 2, B), jnp.float32),
        pltpu.VMEM((16,), jnp.float32),
    ],
)
def _select_kernel(hist_hbm, out_hbm, tmp_v, out_v):
    wid = lax.axis_index("s") * 2 + lax.axis_index("c")

    @pl.when(wid == 0)
    def _():
        pltpu.sync_copy(hist_hbm, tmp_v)
        nk = jnp.float32(NKEEP)
        zeros16 = jnp.zeros((16,), jnp.float32)

        def sel_body(jj, carry):
            cum, accv = carry
            j = (B // 16 - 1) - jj
            sl = pl.ds(j * 16, 16)

            def merge_body(t, cs):
                cvec, svec = cs
                return cvec + tmp_v[t, 0, sl], svec + tmp_v[t, 1, sl]

            cvec, svec = lax.fori_loop(0, NW, merge_body, (zeros16, zeros16))
            sfx = lax.rev(plsc.cumsum(lax.rev(cvec, (0,))), (0,))
            incl = cum + sfx
            above = incl - cvec
            full = incl <= nk
            thr = jnp.logical_and(above < nk, incl > nk)
            safe_c = jnp.maximum(cvec, 1.0)
            meanv = svec / safe_c
            r = nk - above
            vhat = meanv + (1.0 - r / safe_c) * 0.5
            contrib = jnp.where(full, svec, 0.0) + jnp.where(thr, r * vhat, 0.0)
            return (cum + jnp.sum(cvec), accv + contrib)

        _, accv = lax.fori_loop(
            0, B // 16, sel_body, (jnp.float32(0.0), zeros16))
        total = jnp.sum(accv)
        out_v[...] = jnp.zeros((16,), jnp.float32) + total * (
            1.0 / (NKEEP * SCALE))
        pltpu.sync_copy(out_v, out_hbm)


def kernel(prediction, label):
    loss2d = _loss_tc(prediction, label)
    hist = _hist_kernel(loss2d)
    out = _select_kernel(hist)
    return out[0]
